# Initial kernel scaffold; baseline (speedup 1.0000x reference)
import functools

import jax
import jax.numpy as jnp
from jax import lax
from jax.experimental import pallas as pl
from jax.experimental.pallas import tpu as pltpu
from jax.experimental.pallas import tpu_sc as plsc

N = 10000
E = 320000
F = 128
G = 64
NC = 2
NS = 16
NW = NC * NS
EPT = E // NW
K = 100
CH = EPT // K
RPT = N // NS
ZR = 125
DW = 16

_mesh = plsc.VectorSubcoreMesh(
    core_axis_name="c", subcore_axis_name="s", num_cores=NC, num_subcores=NS)

_f32 = jnp.float32


@functools.partial(
    pl.kernel,
    out_type=jax.ShapeDtypeStruct((NC, N, DW), _f32),
    mesh=_mesh,
    scratch_types=[
        pltpu.VMEM((CH, K), jnp.int32),
        pltpu.VMEM((K, DW), _f32),
        pltpu.VMEM((RPT, DW), _f32),
        pltpu.VMEM_SHARED((N, DW), _f32),
        pltpu.SemaphoreType.DMA,
    ],
)
def _deg_kernel(dst_hbm, out_hbm, idxv, onesv, zv, accum, sem):
    c = lax.axis_index("c")
    s = lax.axis_index("s")
    wid = c * NS + s
    zero16 = jnp.zeros((16,), _f32)
    e0 = (lax.broadcasted_iota(jnp.int32, (16,), 0) == 0).astype(_f32)

    def init_rows(i, _):
        zv[i, :] = zero16
        return 0
    lax.fori_loop(0, RPT, init_rows, 0)

    def init_ones(i, _):
        onesv[i, :] = e0
        return 0
    lax.fori_loop(0, K, init_ones, 0)

    pltpu.sync_copy(zv, accum.at[pl.ds(s * RPT, RPT)])
    plsc.subcore_barrier()

    pltpu.sync_copy(dst_hbm.at[wid], idxv)

    def chunk(j, _):
        pltpu.sync_copy(onesv, accum.at[idxv.at[j]], add=True)
        return 0
    lax.fori_loop(0, CH, chunk, 0)

    plsc.subcore_barrier()
    pltpu.sync_copy(accum.at[pl.ds(s * RPT, RPT)],
                    out_hbm.at[c, pl.ds(s * RPT, RPT)])


@functools.partial(
    pl.kernel,
    out_type=jax.ShapeDtypeStruct((NC, N, F), _f32),
    mesh=_mesh,
    scratch_types=[
        pltpu.VMEM((CH, K), jnp.int32),
        pltpu.VMEM((CH, K), jnp.int32),
        pltpu.VMEM((K, F), _f32),
        pltpu.VMEM((ZR, F), _f32),
        pltpu.VMEM_SHARED((N, F), _f32),
        pltpu.SemaphoreType.DMA,
    ],
)
def _agg_kernel(src_hbm, dst_hbm, hp_hbm, out_hbm, srcv, dstv, rows, zv, accum, sem):
    c = lax.axis_index("c")
    s = lax.axis_index("s")
    wid = c * NS + s
    zero16 = jnp.zeros((16,), _f32)

    def zrow(i, _):
        def zcol(j, _):
            zv[i, pl.ds(j * 16, 16)] = zero16
            return 0
        lax.fori_loop(0, F // 16, zcol, 0)
        return 0
    lax.fori_loop(0, ZR, zrow, 0)

    def zcp(t, _):
        pltpu.sync_copy(zv, accum.at[pl.ds(s * RPT + t * ZR, ZR)])
        return 0
    lax.fori_loop(0, RPT // ZR, zcp, 0)
    plsc.subcore_barrier()

    pltpu.sync_copy(src_hbm.at[wid], srcv)
    pltpu.sync_copy(dst_hbm.at[wid], dstv)

    def chunk(j, _):
        pltpu.async_copy(hp_hbm.at[srcv.at[j]], rows, sem).wait()
        pltpu.sync_copy(rows, accum.at[dstv.at[j]], add=True)
        return 0
    lax.fori_loop(0, CH, chunk, 0)

    plsc.subcore_barrier()
    pltpu.sync_copy(accum.at[pl.ds(s * RPT, RPT)],
                    out_hbm.at[c, pl.ds(s * RPT, RPT)])


def _tc_a(x_ref, w1t_ref, degp_ref, hp_ref, dinv_ref):
    t = degp_ref[0] + degp_ref[1]
    deg = 1.0 + jnp.sum(t, axis=1, keepdims=True)
    dinv = lax.rsqrt(deg)
    dinv_ref[...] = dinv
    hp_ref[...] = dinv * jnp.dot(x_ref[...], w1t_ref[...],
                                 preferred_element_type=_f32)


_tc_a_call = pl.pallas_call(
    _tc_a,
    out_shape=(jax.ShapeDtypeStruct((N, F), _f32),
               jax.ShapeDtypeStruct((N, 1), _f32)),
)


def _tc_b(agg_ref, hp_ref, dinv_ref, b_ref, g_ref, be_ref, w2t_ref, out_ref):
    t = dinv_ref[...] * (agg_ref[0] + agg_ref[1] + hp_ref[...]) + b_ref[...]
    mu = jnp.mean(t, axis=0, keepdims=True)
    var = jnp.mean((t - mu) ** 2, axis=0, keepdims=True)
    y = jnp.maximum(g_ref[...] * (t - mu) * lax.rsqrt(var + 1e-5) + be_ref[...],
                    0.0)
    out_ref[...] = dinv_ref[...] * jnp.dot(y, w2t_ref[...],
                                           preferred_element_type=_f32)


_tc_b_call = pl.pallas_call(
    _tc_b, out_shape=jax.ShapeDtypeStruct((N, F), _f32))


def _tc_c(agg_ref, hp_ref, dinv_ref, b_ref, g_ref, be_ref, batch_ref,
          wl1t_ref, bl1_ref, wl2t_ref, bl2_ref, out_ref):
    t = dinv_ref[...] * (agg_ref[0] + agg_ref[1] + hp_ref[...]) + b_ref[...]
    mu = jnp.mean(t, axis=0, keepdims=True)
    var = jnp.mean((t - mu) ** 2, axis=0, keepdims=True)
    y = jnp.maximum(g_ref[...] * (t - mu) * lax.rsqrt(var + 1e-5) + be_ref[...],
                    0.0)
    gid = lax.broadcasted_iota(jnp.int32, (G, N), 0)
    oh = (gid == batch_ref[...]).astype(_f32)
    sums = jnp.dot(oh, y, preferred_element_type=_f32)
    cnt = jnp.sum(oh, axis=1, keepdims=True)
    pooled = sums / jnp.maximum(cnt, 1.0)
    r = jnp.maximum(
        jnp.dot(pooled, wl1t_ref[...], preferred_element_type=_f32)
        + bl1_ref[...], 0.0)
    out_ref[...] = (jnp.dot(r, wl2t_ref[...], preferred_element_type=_f32)
                    + bl2_ref[...])


_tc_c_call = pl.pallas_call(
    _tc_c, out_shape=jax.ShapeDtypeStruct((G, 1), _f32))


def kernel(x, edge_index, batch, W1, b1, g1, be1, W2, b2, g2, be2,
           Wl1, bl1, Wl2, bl2):
    src = edge_index[0].reshape(NW, CH, K)
    dst = edge_index[1].reshape(NW, CH, K)
    degp = _deg_kernel(dst)
    hp1, dinv = _tc_a_call(x, W1.T, degp)
    agg1 = _agg_kernel(src, dst, hp1)
    hp2 = _tc_b_call(agg1, hp1, dinv, b1.reshape(1, F), g1.reshape(1, F),
                     be1.reshape(1, F), W2.T)
    agg2 = _agg_kernel(src, dst, hp2)
    out = _tc_c_call(agg2, hp2, dinv, b2.reshape(1, F), g2.reshape(1, F),
                     be2.reshape(1, F), batch.reshape(1, N), Wl1.T,
                     bl1.reshape(1, G), Wl2.T, bl2.reshape(1, 1))
    return out.reshape(G)


# speedup vs baseline: 9.8347x; 9.8347x over previous
import functools

import jax
import jax.numpy as jnp
from jax import lax
from jax.experimental import pallas as pl
from jax.experimental.pallas import tpu as pltpu
from jax.experimental.pallas import tpu_sc as plsc

N = 10000
E = 320000
F = 128
G = 64
NC = 2
NS = 16
NW = NC * NS
K = 80
CHD = (E // NW) // K
CHA = (E // NS) // K
NP = 10240
RPT = NP // NS
HN = NP // 2
ACC_R = HN + 8
RPA = HN // NS
ZR = 64
DW = 16

_mesh = plsc.VectorSubcoreMesh(
    core_axis_name="c", subcore_axis_name="s", num_cores=NC, num_subcores=NS)

_f32 = jnp.float32


@functools.partial(
    pl.kernel,
    out_type=jax.ShapeDtypeStruct((NC, NP, DW), _f32),
    mesh=_mesh,
    scratch_types=[
        pltpu.VMEM((CHD, K), jnp.int32),
        pltpu.VMEM((16, DW), _f32),
        pltpu.VMEM_SHARED((NP, DW), _f32),
        pltpu.SemaphoreType.DMA,
    ],
)
def _deg_kernel(dst_hbm, zeros_hbm, out_hbm, idxv, onesv, accum, sem):
    c = lax.axis_index("c")
    s = lax.axis_index("s")
    wid = c * NS + s

    def init_ones(i, _):
        onesv[i, :] = jnp.full((16,), 0.0625, _f32)
        return 0
    lax.fori_loop(0, 16, init_ones, 0)

    @pl.when(s == 0)
    def _():
        pltpu.sync_copy(zeros_hbm, accum)
    plsc.subcore_barrier()

    pltpu.sync_copy(dst_hbm.at[wid], idxv)

    def chunk(j, _):
        for q in range(K // 16):
            idx16 = idxv[j, pl.ds(q * 16, 16)]
            pltpu.sync_copy(onesv, accum.at[idx16], add=True)
        return 0
    lax.fori_loop(0, CHD, chunk, 0)

    plsc.subcore_barrier()

    @pl.when(s == 0)
    def _():
        pltpu.sync_copy(accum, out_hbm.at[c])


@functools.partial(
    pl.kernel,
    out_type=jax.ShapeDtypeStruct((NC, ACC_R, F), _f32),
    mesh=_mesh,
    scratch_types=[
        pltpu.VMEM((CHA, K), jnp.int32),
        pltpu.VMEM((CHA, K), jnp.int32),
        pltpu.VMEM((K, F), _f32),
        pltpu.VMEM_SHARED((ACC_R, F), _f32),
        pltpu.SemaphoreType.DMA,
    ],
)
def _agg_kernel(src_hbm, rdst_hbm, hp_hbm, zeros_hbm, out_hbm,
                srcv, dstv, rows, accum, sem):
    c = lax.axis_index("c")
    s = lax.axis_index("s")

    @pl.when(s == 0)
    def _():
        pltpu.sync_copy(zeros_hbm, accum)
    plsc.subcore_barrier()

    pltpu.sync_copy(src_hbm.at[s], srcv)
    pltpu.sync_copy(rdst_hbm.at[c, s], dstv)

    def chunk(j, _):
        pltpu.async_copy(hp_hbm.at[srcv.at[j]], rows, sem).wait()
        for q in range(K // 16):
            idx16 = dstv[j, pl.ds(q * 16, 16)]
            pltpu.sync_copy(rows.at[pl.ds(q * 16, 16)], accum.at[idx16],
                            add=True)
        return 0
    lax.fori_loop(0, CHA, chunk, 0)

    plsc.subcore_barrier()

    @pl.when(s == 0)
    def _():
        pltpu.sync_copy(accum, out_hbm.at[c])


def _tc_a(x_ref, w1t_ref, degp_ref, dst_ref, hp_ref, dinv_ref, rdst_ref):
    t = degp_ref[0, :N] + degp_ref[1, :N]
    deg = 1.0 + jnp.sum(t, axis=1, keepdims=True)
    dinv = lax.rsqrt(deg)
    dinv_ref[...] = dinv
    hp_ref[...] = dinv * jnp.dot(x_ref[...], w1t_ref[...],
                                 preferred_element_type=_f32)
    d = dst_ref[...]
    rdst_ref[0] = jnp.where(d < HN, d, HN)
    d1 = d - HN
    rdst_ref[1] = jnp.where(d1 >= 0, d1, HN)


_tc_a_call = pl.pallas_call(
    _tc_a,
    out_shape=(jax.ShapeDtypeStruct((N, F), _f32),
               jax.ShapeDtypeStruct((N, 1), _f32),
               jax.ShapeDtypeStruct((NC, NS, CHA, K), jnp.int32)),
)


def _stitch(agg_ref):
    return jnp.concatenate([agg_ref[0, :HN], agg_ref[1, :N - HN]], axis=0)


def _bn_relu(agg, hp, dinv, b, g, be):
    t = dinv * (agg + hp) + b
    mu = jnp.mean(t, axis=0, keepdims=True)
    var = jnp.mean((t - mu) ** 2, axis=0, keepdims=True)
    return jnp.maximum(g * (t - mu) * lax.rsqrt(var + 1e-5) + be, 0.0)


def _tc_b(agg_ref, hp_ref, dinv_ref, b_ref, g_ref, be_ref, w2t_ref, out_ref):
    dinv = dinv_ref[...]
    y = _bn_relu(_stitch(agg_ref), hp_ref[...], dinv,
                 b_ref[...], g_ref[...], be_ref[...])
    out_ref[...] = dinv * jnp.dot(y, w2t_ref[...], preferred_element_type=_f32)


_tc_b_call = pl.pallas_call(
    _tc_b, out_shape=jax.ShapeDtypeStruct((N, F), _f32))


def _tc_c(agg_ref, hp_ref, dinv_ref, b_ref, g_ref, be_ref,
          batch_ref, wl1t_ref, bl1_ref, wl2t_ref, bl2_ref, out_ref):
    dinv = dinv_ref[...]
    y = _bn_relu(_stitch(agg_ref), hp_ref[...], dinv,
                 b_ref[...], g_ref[...], be_ref[...])
    gid = lax.broadcasted_iota(jnp.int32, (G, N), 0)
    oh = (gid == batch_ref[...]).astype(_f32)
    cnt = jnp.maximum(jnp.sum(oh, axis=1, keepdims=True), 1.0)
    pooled = jnp.dot(oh, y, preferred_element_type=_f32) / cnt
    r = jnp.maximum(
        jnp.dot(pooled, wl1t_ref[...], preferred_element_type=_f32)
        + bl1_ref[...], 0.0)
    out_ref[...] = (jnp.dot(r, wl2t_ref[...], preferred_element_type=_f32)
                    + bl2_ref[...])


_tc_c_call = pl.pallas_call(
    _tc_c, out_shape=jax.ShapeDtypeStruct((G, 1), _f32))


def kernel(x, edge_index, batch, W1, b1, g1, be1, W2, b2, g2, be2,
           Wl1, bl1, Wl2, bl2):
    src = edge_index[0].reshape(NS, CHA, K)
    dst = edge_index[1].reshape(NS, CHA, K)
    degp = _deg_kernel(edge_index[1].reshape(NW, CHD, K),
                       jnp.zeros((NP, DW), _f32))
    hp1, dinv, rdst = _tc_a_call(x, W1.T, degp, dst)
    az = jnp.zeros((ACC_R, F), _f32)
    agg1 = _agg_kernel(src, rdst, hp1, az)
    hp2 = _tc_b_call(agg1, hp1, dinv, b1.reshape(1, F), g1.reshape(1, F),
                     be1.reshape(1, F), W2.T)
    agg2 = _agg_kernel(src, rdst, hp2, az)
    out = _tc_c_call(agg2, hp2, dinv, b2.reshape(1, F), g2.reshape(1, F),
                     be2.reshape(1, F), batch.reshape(1, N), Wl1.T,
                     bl1.reshape(1, G), Wl2.T, bl2.reshape(1, 1))
    return out.reshape(G)


# speedup vs baseline: 16.6865x; 1.6967x over previous
import functools

import jax
import jax.numpy as jnp
from jax import lax
from jax.experimental import pallas as pl
from jax.experimental.pallas import tpu as pltpu
from jax.experimental.pallas import tpu_sc as plsc

N = 10000
E = 320000
F = 128
G = 64
NC = 2
NS = 16
NW = NC * NS
K = 80
CHD = (E // NW) // K
CHA = (E // NS) // K
NP = 10240
RPT = NP // NS
HN = NP // 2
ACC_R = HN + 8
RPA = HN // NS
ZR = 64
DW = 16

_mesh = plsc.VectorSubcoreMesh(
    core_axis_name="c", subcore_axis_name="s", num_cores=NC, num_subcores=NS)

_f32 = jnp.float32


@functools.partial(
    pl.kernel,
    out_type=jax.ShapeDtypeStruct((NC, NP, DW), _f32),
    mesh=_mesh,
    scratch_types=[
        pltpu.VMEM((CHD, K), jnp.int32),
        pltpu.VMEM((16, DW), _f32),
        pltpu.VMEM_SHARED((NP, DW), _f32),
        pltpu.SemaphoreType.DMA,
    ],
)
def _deg_kernel(dst_hbm, zeros_hbm, out_hbm, idxv, onesv, accum, sem):
    c = lax.axis_index("c")
    s = lax.axis_index("s")
    wid = c * NS + s

    def init_ones(i, _):
        onesv[i, :] = jnp.full((16,), 0.0625, _f32)
        return 0
    lax.fori_loop(0, 16, init_ones, 0)

    @pl.when(s == 0)
    def _():
        pltpu.sync_copy(zeros_hbm, accum)
    plsc.subcore_barrier()

    pltpu.sync_copy(dst_hbm.at[wid], idxv)

    def chunk(j, _):
        cps = []
        for q in range(K // 16):
            idx16 = idxv[j, pl.ds(q * 16, 16)]
            cps.append(pltpu.async_copy(onesv, accum.at[idx16], sem,
                                        add=True))
        for cp in cps:
            cp.wait()
        return 0
    lax.fori_loop(0, CHD, chunk, 0)

    plsc.subcore_barrier()

    @pl.when(s == 0)
    def _():
        pltpu.sync_copy(accum, out_hbm.at[c])


@functools.partial(
    pl.kernel,
    out_type=jax.ShapeDtypeStruct((NC, ACC_R, F), _f32),
    mesh=_mesh,
    scratch_types=[
        pltpu.VMEM((CHA, K), jnp.int32),
        pltpu.VMEM((CHA, K), jnp.int32),
        pltpu.VMEM((K, F), _f32),
        pltpu.VMEM((K, F), _f32),
        pltpu.VMEM_SHARED((ACC_R, F), _f32),
        pltpu.SemaphoreType.DMA,
        pltpu.SemaphoreType.DMA,
        pltpu.SemaphoreType.DMA,
        pltpu.SemaphoreType.DMA,
    ],
)
def _agg_kernel(src_hbm, rdst_hbm, hp_hbm, zeros_hbm, out_hbm,
                srcv, dstv, rows0, rows1, accum, gsem0, gsem1, ssem0, ssem1):
    c = lax.axis_index("c")
    s = lax.axis_index("s")

    @pl.when(s == 0)
    def _():
        pltpu.sync_copy(zeros_hbm, accum)
    plsc.subcore_barrier()

    pltpu.sync_copy(src_hbm.at[s], srcv)
    pltpu.sync_copy(rdst_hbm.at[c, s], dstv)

    def scat(rows, j, ssem):
        cps = []
        for q in range(K // 16):
            idx16 = dstv[j, pl.ds(q * 16, 16)]
            cps.append(pltpu.async_copy(rows.at[pl.ds(q * 16, 16)],
                                        accum.at[idx16], ssem, add=True))
        for cp in cps:
            cp.wait()

    pltpu.async_copy(hp_hbm.at[srcv.at[0]], rows0, gsem0)

    def pair(t, _):
        j0 = 2 * t
        j1 = 2 * t + 1
        pltpu.async_copy(hp_hbm.at[srcv.at[j1]], rows1, gsem1)
        pltpu.make_async_copy(hp_hbm.at[srcv.at[j0]], rows0, gsem0).wait()
        scat(rows0, j0, ssem0)

        @pl.when(j0 + 2 < CHA)
        def _():
            pltpu.async_copy(hp_hbm.at[srcv.at[j0 + 2]], rows0, gsem0)
        pltpu.make_async_copy(hp_hbm.at[srcv.at[j1]], rows1, gsem1).wait()
        scat(rows1, j1, ssem1)
        return 0
    lax.fori_loop(0, CHA // 2, pair, 0)

    plsc.subcore_barrier()

    @pl.when(s == 0)
    def _():
        pltpu.sync_copy(accum, out_hbm.at[c])


def _tc_a(x_ref, w1t_ref, degp_ref, dst_ref, hp_ref, dinv_ref, rdst_ref):
    t = degp_ref[0, :N] + degp_ref[1, :N]
    deg = 1.0 + jnp.sum(t, axis=1, keepdims=True)
    dinv = lax.rsqrt(deg)
    dinv_ref[...] = dinv
    hp_ref[...] = dinv * jnp.dot(x_ref[...], w1t_ref[...],
                                 preferred_element_type=_f32)
    d = dst_ref[...]
    rdst_ref[0] = jnp.where(d < HN, d, HN)
    d1 = d - HN
    rdst_ref[1] = jnp.where(d1 >= 0, d1, HN)


_tc_a_call = pl.pallas_call(
    _tc_a,
    out_shape=(jax.ShapeDtypeStruct((N, F), _f32),
               jax.ShapeDtypeStruct((N, 1), _f32),
               jax.ShapeDtypeStruct((NC, NS, CHA, K), jnp.int32)),
)


def _stitch(agg_ref):
    return jnp.concatenate([agg_ref[0, :HN], agg_ref[1, :N - HN]], axis=0)


def _bn_relu(agg, hp, dinv, b, g, be):
    t = dinv * (agg + hp) + b
    mu = jnp.mean(t, axis=0, keepdims=True)
    var = jnp.mean((t - mu) ** 2, axis=0, keepdims=True)
    return jnp.maximum(g * (t - mu) * lax.rsqrt(var + 1e-5) + be, 0.0)


def _tc_b(agg_ref, hp_ref, dinv_ref, b_ref, g_ref, be_ref, w2t_ref, out_ref):
    dinv = dinv_ref[...]
    y = _bn_relu(_stitch(agg_ref), hp_ref[...], dinv,
                 b_ref[...], g_ref[...], be_ref[...])
    out_ref[...] = dinv * jnp.dot(y, w2t_ref[...], preferred_element_type=_f32)


_tc_b_call = pl.pallas_call(
    _tc_b, out_shape=jax.ShapeDtypeStruct((N, F), _f32))


def _tc_c(agg_ref, hp_ref, dinv_ref, b_ref, g_ref, be_ref,
          batch_ref, wl1t_ref, bl1_ref, wl2t_ref, bl2_ref, out_ref):
    dinv = dinv_ref[...]
    y = _bn_relu(_stitch(agg_ref), hp_ref[...], dinv,
                 b_ref[...], g_ref[...], be_ref[...])
    gid = lax.broadcasted_iota(jnp.int32, (G, N), 0)
    oh = (gid == batch_ref[...]).astype(_f32)
    cnt = jnp.maximum(jnp.sum(oh, axis=1, keepdims=True), 1.0)
    pooled = jnp.dot(oh, y, preferred_element_type=_f32) / cnt
    r = jnp.maximum(
        jnp.dot(pooled, wl1t_ref[...], preferred_element_type=_f32)
        + bl1_ref[...], 0.0)
    out_ref[...] = (jnp.dot(r, wl2t_ref[...], preferred_element_type=_f32)
                    + bl2_ref[...])


_tc_c_call = pl.pallas_call(
    _tc_c, out_shape=jax.ShapeDtypeStruct((G, 1), _f32))


def kernel(x, edge_index, batch, W1, b1, g1, be1, W2, b2, g2, be2,
           Wl1, bl1, Wl2, bl2):
    src = edge_index[0].reshape(NS, CHA, K)
    dst = edge_index[1].reshape(NS, CHA, K)
    degp = _deg_kernel(edge_index[1].reshape(NW, CHD, K),
                       jnp.zeros((NP, DW), _f32))
    hp1, dinv, rdst = _tc_a_call(x, W1.T, degp, dst)
    az = jnp.zeros((ACC_R, F), _f32)
    agg1 = _agg_kernel(src, rdst, hp1, az)
    hp2 = _tc_b_call(agg1, hp1, dinv, b1.reshape(1, F), g1.reshape(1, F),
                     be1.reshape(1, F), W2.T)
    agg2 = _agg_kernel(src, rdst, hp2, az)
    out = _tc_c_call(agg2, hp2, dinv, b2.reshape(1, F), g2.reshape(1, F),
                     be2.reshape(1, F), batch.reshape(1, N), Wl1.T,
                     bl1.reshape(1, G), Wl2.T, bl2.reshape(1, 1))
    return out.reshape(G)
